# baseline (device time: 73766 ns/iter reference)
import jax
import jax.numpy as jnp
from jax import lax
from jax.experimental import pallas as pl
from jax.experimental.pallas import tpu as pltpu

N_DEV = 4
B, SQ, SKV, HQ, DH = 4, 256, 4096, 32, 128
H_LOC = HQ // N_DEV
DM = 1024
QBLK = 64
NQB = SQ // QBLK
STRIDE = 4
NT = SKV // (QBLK * STRIDE)
SCALE = 0.08838834764831843


def _fused(mp, x, Wq, K_r, V_r, Wo):
    def body(mp_ref, x_ref, wq_ref, k_ref, v_ref, wo_ref, out_ref,
             xg, q_scr, o_rem, o_recv, wq_s, wo_s,
             x_send, x_recv, o_send, o_recv_sem, w_sem):
        s = pl.program_id(0)
        c = pl.program_id(1)
        my = mp_ref[0]

        @pl.when((s == 0) & (c == 0))
        def _start():
            cwq = pltpu.make_async_copy(wq_ref, wq_s, w_sem.at[0])
            cwo = pltpu.make_async_copy(wo_ref, wo_s, w_sem.at[1])
            cwq.start()
            cwo.start()
            barrier = pltpu.get_barrier_semaphore()
            for d in range(1, N_DEV):
                pl.semaphore_signal(
                    barrier, inc=1, device_id=((my + d) % N_DEV,),
                    device_id_type=pl.DeviceIdType.MESH,
                )
            pl.semaphore_wait(barrier, N_DEV - 1)
            xg[pl.ds(0, 1), :, :] = x_ref[...].astype(jnp.bfloat16)
            for d in (3, 2, 1):
                pltpu.make_async_remote_copy(
                    src_ref=xg.at[pl.ds(0, 1)],
                    dst_ref=xg.at[pl.ds(N_DEV - d, 1)],
                    send_sem=x_send.at[d - 1],
                    recv_sem=x_recv.at[N_DEV - d - 1],
                    device_id=((my + d) % N_DEV,),
                    device_id_type=pl.DeviceIdType.MESH,
                ).start()
            cwq.wait()
            cwo.wait()

        @pl.when((s >= 1) & (c == 0))
        def _wait_x():
            pltpu.make_async_remote_copy(
                src_ref=xg.at[pl.ds(0, 1)],
                dst_ref=xg.at[pl.ds(s, 1)],
                send_sem=x_send.at[0],
                recv_sem=x_recv.at[s - 1],
                device_id=(my,),
                device_id_type=pl.DeviceIdType.MESH,
            ).wait_recv()

        @pl.when(c == 0)
        def _proj_q():
            q_scr[...] = jnp.dot(
                xg[pl.ds(s, 1), :, :][0].astype(jnp.float32), wq_s[...],
                preferred_element_type=jnp.float32,
            )

        q = q_scr[pl.ds(c * QBLK, QBLK), :].reshape(QBLK, H_LOC, DH)
        k = k_ref[0, :, 0].reshape(NT * QBLK, H_LOC, DH)
        v = v_ref[0, :, 0].reshape(NT * QBLK, H_LOC, DH)
        qb = q.transpose(1, 0, 2)
        kb = k.transpose(1, 0, 2)
        vb = v.transpose(1, 0, 2)
        sc = lax.dot_general(
            qb, kb, (((2,), (2,)), ((0,), (0,))),
            preferred_element_type=jnp.float32,
        ) * SCALE
        w = jnp.exp(sc)
        denom = jnp.sum(w, axis=-1, keepdims=True)
        ctxb = lax.dot_general(
            w, vb, (((2,), (1,)), ((0,), (0,))),
            preferred_element_type=jnp.float32,
        ) / denom
        ctx = ctxb.transpose(1, 0, 2).reshape(QBLK, DM)
        part = jnp.dot(ctx, wo_s[...],
                       preferred_element_type=jnp.float32)

        @pl.when(s == 0)
        def _own():
            out_ref[0, pl.ds(c * QBLK, QBLK), :] = part

        @pl.when(s >= 1)
        def _send_part():
            o_rem[pl.ds(s - 1, 1), pl.ds(c * QBLK, QBLK), :] = (
                part[None].astype(jnp.bfloat16))
            pltpu.make_async_remote_copy(
                src_ref=o_rem.at[pl.ds(s - 1, 1), pl.ds(c * QBLK, QBLK)],
                dst_ref=o_recv.at[pl.ds(N_DEV - 1 - s, 1),
                                  pl.ds(c * QBLK, QBLK)],
                send_sem=o_send.at[(s - 1) * NQB + c],
                recv_sem=o_recv_sem.at[(N_DEV - 1 - s) * NQB + c],
                device_id=((my + s) % N_DEV,),
                device_id_type=pl.DeviceIdType.MESH,
            ).start()

        @pl.when((s == N_DEV - 1) & (c == NQB - 1))
        def _finish():
            for k_ in range(N_DEV - 1):
                for c_ in range(NQB):
                    pltpu.make_async_remote_copy(
                        src_ref=o_rem.at[pl.ds(0, 1),
                                         pl.ds(c_ * QBLK, QBLK)],
                        dst_ref=o_recv.at[pl.ds(k_, 1),
                                          pl.ds(c_ * QBLK, QBLK)],
                        send_sem=o_send.at[0],
                        recv_sem=o_recv_sem.at[k_ * NQB + c_],
                        device_id=(my,),
                        device_id_type=pl.DeviceIdType.MESH,
                    ).wait_recv()
            out_ref[0] = (out_ref[0]
                          + o_recv[0].astype(jnp.float32)
                          + o_recv[1].astype(jnp.float32)
                          + o_recv[2].astype(jnp.float32))
            for k_ in range(N_DEV - 1):
                pltpu.make_async_remote_copy(
                    src_ref=xg.at[pl.ds(0, 1)],
                    dst_ref=xg.at[pl.ds(0, 1)],
                    send_sem=x_send.at[k_],
                    recv_sem=x_recv.at[0],
                    device_id=(my,),
                    device_id_type=pl.DeviceIdType.MESH,
                ).wait_send()
                for c_ in range(NQB):
                    pltpu.make_async_remote_copy(
                        src_ref=o_rem.at[pl.ds(k_, 1),
                                         pl.ds(c_ * QBLK, QBLK)],
                        dst_ref=o_recv.at[pl.ds(0, 1),
                                          pl.ds(c_ * QBLK, QBLK)],
                        send_sem=o_send.at[k_ * NQB + c_],
                        recv_sem=o_recv_sem.at[0],
                        device_id=(my,),
                        device_id_type=pl.DeviceIdType.MESH,
                    ).wait_send()

    grid_spec = pltpu.PrefetchScalarGridSpec(
        num_scalar_prefetch=1,
        grid=(N_DEV, NQB),
        in_specs=[
            pl.BlockSpec((1, SQ, DM), lambda s, c, mp: (0, 0, 0)),
            pl.BlockSpec(memory_space=pl.ANY),
            pl.BlockSpec((1, NT, 1, QBLK, H_LOC, DH),
                         lambda s, c, mp: ((mp[0] + s) % N_DEV, 0, c, 0,
                                           mp[0], 0)),
            pl.BlockSpec((1, NT, 1, QBLK, H_LOC, DH),
                         lambda s, c, mp: ((mp[0] + s) % N_DEV, 0, c, 0,
                                           mp[0], 0)),
            pl.BlockSpec(memory_space=pl.ANY),
        ],
        out_specs=pl.BlockSpec((1, SQ, DM), lambda s, c, mp: (0, 0, 0)),
        scratch_shapes=[
            pltpu.VMEM((N_DEV, SQ, DM), jnp.bfloat16),
            pltpu.VMEM((SQ, DM), jnp.float32),
            pltpu.VMEM((N_DEV - 1, SQ, DM), jnp.bfloat16),
            pltpu.VMEM((N_DEV - 1, SQ, DM), jnp.bfloat16),
            pltpu.VMEM((DM, DM), jnp.float32),
            pltpu.VMEM((DM, DM), jnp.float32),
            pltpu.SemaphoreType.DMA((N_DEV - 1,)),
            pltpu.SemaphoreType.DMA((N_DEV - 1,)),
            pltpu.SemaphoreType.DMA(((N_DEV - 1) * NQB,)),
            pltpu.SemaphoreType.DMA(((N_DEV - 1) * NQB,)),
            pltpu.SemaphoreType.DMA((2,)),
        ],
    )
    return pl.pallas_call(
        body,
        grid_spec=grid_spec,
        out_shape=jax.ShapeDtypeStruct((1, SQ, DM), jnp.float32),
        compiler_params=pltpu.CompilerParams(
            collective_id=0,
            dimension_semantics=("arbitrary", "arbitrary"),
            vmem_limit_bytes=40 * 1024 * 1024,
        ),
    )(mp, x, Wq, K_r, V_r, Wo)


def kernel(x, Wq, K_ext, V_ext, Wo):
    my = lax.axis_index("i")
    K_r = K_ext.reshape(B, NT, STRIDE, QBLK, HQ, DH)
    V_r = V_ext.reshape(B, NT, STRIDE, QBLK, HQ, DH)
    mp = jnp.full((1,), my, dtype=jnp.int32)
    return _fused(mp, x, Wq, K_r, V_r, Wo)


# device time: 72188 ns/iter; 1.0219x vs baseline; 1.0219x over previous
import jax
import jax.numpy as jnp
from jax import lax
from jax.experimental import pallas as pl
from jax.experimental.pallas import tpu as pltpu

N_DEV = 4
B, SQ, SKV, HQ, DH = 4, 256, 4096, 32, 128
H_LOC = HQ // N_DEV
DM = 1024
QBLK = 64
NQB = SQ // QBLK
STRIDE = 4
NT = SKV // (QBLK * STRIDE)
SCALE = 0.08838834764831843


def _fused(mp, x, Wq, K_r, V_r, Wo):
    def body(mp_ref, x_ref, wq_ref, k_ref, v_ref, wo_ref, out_ref,
             xg, q_scr, o_rem, o_recv, wq_s, wo_s,
             x_send, x_recv, o_send, o_recv_sem, w_sem):
        s = pl.program_id(0)
        c = pl.program_id(1)
        my = mp_ref[0]

        @pl.when((s == 0) & (c == 0))
        def _start():
            cwq = pltpu.make_async_copy(wq_ref, wq_s, w_sem.at[0])
            cwo = pltpu.make_async_copy(wo_ref, wo_s, w_sem.at[1])
            cwq.start()
            cwo.start()
            barrier = pltpu.get_barrier_semaphore()
            for d in range(1, N_DEV):
                pl.semaphore_signal(
                    barrier, inc=1, device_id=((my + d) % N_DEV,),
                    device_id_type=pl.DeviceIdType.MESH,
                )
            pl.semaphore_wait(barrier, N_DEV - 1)
            xg[pl.ds(0, 1), :, :] = x_ref[...].astype(jnp.bfloat16)
            for d in (3, 2, 1):
                pltpu.make_async_remote_copy(
                    src_ref=xg.at[pl.ds(0, 1)],
                    dst_ref=xg.at[pl.ds(N_DEV - d, 1)],
                    send_sem=x_send.at[d - 1],
                    recv_sem=x_recv.at[N_DEV - d - 1],
                    device_id=((my + d) % N_DEV,),
                    device_id_type=pl.DeviceIdType.MESH,
                ).start()
            cwq.wait()
            cwo.wait()

        @pl.when((s >= 1) & (c == 0))
        def _wait_x():
            pltpu.make_async_remote_copy(
                src_ref=xg.at[pl.ds(0, 1)],
                dst_ref=xg.at[pl.ds(s, 1)],
                send_sem=x_send.at[0],
                recv_sem=x_recv.at[s - 1],
                device_id=(my,),
                device_id_type=pl.DeviceIdType.MESH,
            ).wait_recv()

        @pl.when(c == 0)
        def _proj_q():
            q_scr[...] = jnp.dot(
                xg[pl.ds(s, 1), :, :][0].astype(jnp.float32), wq_s[...],
                preferred_element_type=jnp.float32,
            )

        q = q_scr[pl.ds(c * QBLK, QBLK), :].reshape(QBLK, H_LOC, DH)
        k = k_ref[0, :, 0].reshape(NT * QBLK, H_LOC, DH)
        v = v_ref[0, :, 0].reshape(NT * QBLK, H_LOC, DH)
        qb = q.transpose(1, 0, 2)
        kb = k.transpose(1, 0, 2)
        vb = v.transpose(1, 0, 2)
        sc = lax.dot_general(
            qb, kb, (((2,), (2,)), ((0,), (0,))),
            preferred_element_type=jnp.float32,
        ) * SCALE
        w = jnp.exp(sc)
        denom = jnp.sum(w, axis=-1, keepdims=True)
        ctxb = lax.dot_general(
            w, vb, (((2,), (1,)), ((0,), (0,))),
            preferred_element_type=jnp.float32,
        ) / denom
        ctx = ctxb.transpose(1, 0, 2).reshape(QBLK, DM)
        part = jnp.dot(ctx, wo_s[...],
                       preferred_element_type=jnp.float32)

        @pl.when(s == 0)
        def _own():
            out_ref[0, pl.ds(c * QBLK, QBLK), :] = part

        @pl.when(s >= 1)
        def _send_part():
            o_rem[pl.ds(s - 1, 1), pl.ds(c * QBLK, QBLK), :] = (
                part[None].astype(jnp.bfloat16))
            pltpu.make_async_remote_copy(
                src_ref=o_rem.at[pl.ds(s - 1, 1), pl.ds(c * QBLK, QBLK)],
                dst_ref=o_recv.at[pl.ds(N_DEV - 1 - s, 1),
                                  pl.ds(c * QBLK, QBLK)],
                send_sem=o_send.at[(s - 1) * NQB + c],
                recv_sem=o_recv_sem.at[(N_DEV - 1 - s) * NQB + c],
                device_id=((my + s) % N_DEV,),
                device_id_type=pl.DeviceIdType.MESH,
            ).start()

        @pl.when((s == N_DEV - 1) & (c == NQB - 1))
        def _finish():
            for k_ in range(N_DEV - 1):
                for c_ in range(NQB):
                    pltpu.make_async_remote_copy(
                        src_ref=o_rem.at[pl.ds(0, 1),
                                         pl.ds(c_ * QBLK, QBLK)],
                        dst_ref=o_recv.at[pl.ds(k_, 1),
                                          pl.ds(c_ * QBLK, QBLK)],
                        send_sem=o_send.at[0],
                        recv_sem=o_recv_sem.at[k_ * NQB + c_],
                        device_id=(my,),
                        device_id_type=pl.DeviceIdType.MESH,
                    ).wait_recv()
            out_ref[0] = (out_ref[0]
                          + o_recv[0].astype(jnp.float32)
                          + o_recv[1].astype(jnp.float32)
                          + o_recv[2].astype(jnp.float32))
            for k_ in range(N_DEV - 1):
                pltpu.make_async_remote_copy(
                    src_ref=xg.at[pl.ds(0, 1)],
                    dst_ref=xg.at[pl.ds(0, 1)],
                    send_sem=x_send.at[k_],
                    recv_sem=x_recv.at[0],
                    device_id=(my,),
                    device_id_type=pl.DeviceIdType.MESH,
                ).wait_send()
                for c_ in range(NQB):
                    pltpu.make_async_remote_copy(
                        src_ref=o_rem.at[pl.ds(k_, 1),
                                         pl.ds(c_ * QBLK, QBLK)],
                        dst_ref=o_recv.at[pl.ds(0, 1),
                                          pl.ds(c_ * QBLK, QBLK)],
                        send_sem=o_send.at[k_ * NQB + c_],
                        recv_sem=o_recv_sem.at[0],
                        device_id=(my,),
                        device_id_type=pl.DeviceIdType.MESH,
                    ).wait_send()

    grid_spec = pltpu.PrefetchScalarGridSpec(
        num_scalar_prefetch=1,
        grid=(N_DEV, NQB),
        in_specs=[
            pl.BlockSpec((1, SQ, DM), lambda s, c, mp: (0, 0, 0)),
            pl.BlockSpec(memory_space=pl.ANY),
            pl.BlockSpec((1, NT, 1, QBLK, H_LOC, DH),
                         lambda s, c, mp: ((mp[0] + s) % N_DEV, 0, c, 0,
                                           mp[0], 0)),
            pl.BlockSpec((1, NT, 1, QBLK, H_LOC, DH),
                         lambda s, c, mp: ((mp[0] + s) % N_DEV, 0, c, 0,
                                           mp[0], 0)),
            pl.BlockSpec(memory_space=pl.ANY),
        ],
        out_specs=pl.BlockSpec((1, SQ, DM), lambda s, c, mp: (0, 0, 0)),
        scratch_shapes=[
            pltpu.VMEM((N_DEV, SQ, DM), jnp.bfloat16),
            pltpu.VMEM((SQ, DM), jnp.float32),
            pltpu.VMEM((N_DEV - 1, SQ, DM), jnp.bfloat16),
            pltpu.VMEM((N_DEV - 1, SQ, DM), jnp.bfloat16),
            pltpu.VMEM((DM, DM), jnp.float32),
            pltpu.VMEM((DM, DM), jnp.float32),
            pltpu.SemaphoreType.DMA((N_DEV - 1,)),
            pltpu.SemaphoreType.DMA((N_DEV - 1,)),
            pltpu.SemaphoreType.DMA(((N_DEV - 1) * NQB,)),
            pltpu.SemaphoreType.DMA(((N_DEV - 1) * NQB,)),
            pltpu.SemaphoreType.DMA((2,)),
        ],
    )
    return pl.pallas_call(
        body,
        grid_spec=grid_spec,
        out_shape=jax.ShapeDtypeStruct((1, SQ, DM), jnp.float32),
        compiler_params=pltpu.CompilerParams(
            collective_id=0,
            dimension_semantics=("arbitrary", "arbitrary"),
            vmem_limit_bytes=40 * 1024 * 1024,
        ),
    )(mp, x,
      pltpu.with_memory_space_constraint(Wq, pltpu.MemorySpace.HBM),
      K_r, V_r,
      pltpu.with_memory_space_constraint(Wo, pltpu.MemorySpace.HBM))


def kernel(x, Wq, K_ext, V_ext, Wo):
    my = lax.axis_index("i")
    K_r = K_ext.reshape(B, NT, STRIDE, QBLK, HQ, DH)
    V_r = V_ext.reshape(B, NT, STRIDE, QBLK, HQ, DH)
    mp = jnp.full((1,), my, dtype=jnp.int32)
    return _fused(mp, x, Wq, K_r, V_r, Wo)
